# Initial kernel scaffold; baseline (speedup 1.0000x reference)
import jax
import jax.numpy as jnp
from jax.experimental import pallas as pl


def kernel(next_id, next_category, next_subcategory, id_table, category_table, subcategory_table):
    raise NotImplementedError("write your pallas kernel here")


# speedup vs baseline: 1.1524x; 1.1524x over previous
import functools

import jax
import jax.numpy as jnp
from jax import lax
from jax.experimental import pallas as pl
from jax.experimental.pallas import tpu as pltpu
from jax.experimental.pallas import tpu_sc as plsc

EMBED = 64


def kernel(next_id, next_category, next_subcategory, id_table, category_table,
           subcategory_table):
    B = next_id.shape[0]
    next_id = next_id.astype(jnp.int32)
    next_category = next_category.astype(jnp.int32)
    next_subcategory = next_subcategory.astype(jnp.int32)

    info = plsc.get_sparse_core_info()
    nw = info.num_cores * info.num_subcores
    b_per_w = B // nw

    mesh = plsc.VectorSubcoreMesh(core_axis_name="c", subcore_axis_name="s")

    @functools.partial(
        pl.kernel,
        mesh=mesh,
        out_type=jax.ShapeDtypeStruct((B, 3 * EMBED), jnp.float32),
        compiler_params=pltpu.CompilerParams(use_tc_tiling_on_sc=False),
        scratch_types=[
            pltpu.VMEM((b_per_w,), jnp.int32),
            pltpu.VMEM((b_per_w,), jnp.int32),
            pltpu.VMEM((b_per_w,), jnp.int32),
            pltpu.VMEM((b_per_w, EMBED), jnp.float32),
            pltpu.VMEM((b_per_w, EMBED), jnp.float32),
            pltpu.VMEM((b_per_w, EMBED), jnp.float32),
            pltpu.SemaphoreType.DMA,
            pltpu.SemaphoreType.DMA,
            pltpu.SemaphoreType.DMA,
        ],
    )
    def gather_concat(id_idx_hbm, cat_idx_hbm, sub_idx_hbm, id_tab, cat_tab,
                      sub_tab, out_hbm, idx0, idx1, idx2, rows0, rows1, rows2,
                      sem0, sem1, sem2):
        wid = lax.axis_index("s") * info.num_cores + lax.axis_index("c")
        base = wid * b_per_w
        pltpu.sync_copy(id_idx_hbm.at[pl.ds(base, b_per_w)], idx0)
        pltpu.sync_copy(cat_idx_hbm.at[pl.ds(base, b_per_w)], idx1)
        pltpu.sync_copy(sub_idx_hbm.at[pl.ds(base, b_per_w)], idx2)
        c0 = pltpu.async_copy(id_tab.at[idx0], rows0, sem0)
        c1 = pltpu.async_copy(cat_tab.at[idx1], rows1, sem1)
        c2 = pltpu.async_copy(sub_tab.at[idx2], rows2, sem2)
        c0.wait()
        pltpu.sync_copy(rows0, out_hbm.at[pl.ds(base, b_per_w), pl.ds(0, EMBED)])
        c1.wait()
        pltpu.sync_copy(rows1, out_hbm.at[pl.ds(base, b_per_w), pl.ds(EMBED, EMBED)])
        c2.wait()
        pltpu.sync_copy(rows2, out_hbm.at[pl.ds(base, b_per_w), pl.ds(2 * EMBED, EMBED)])

    return gather_concat(next_id, next_category, next_subcategory, id_table,
                         category_table, subcategory_table)
